# Initial kernel scaffold; baseline (speedup 1.0000x reference)
import jax
import jax.numpy as jnp
from jax.experimental import pallas as pl


def kernel(inp, emb):
    raise NotImplementedError("write your pallas kernel here")


# speedup vs baseline: 1.2179x; 1.2179x over previous
import jax
import jax.numpy as jnp
from jax.experimental import pallas as pl

B = 4
S = 8192
D = 768
S_BLK = 512


def _body(inp_ref, emb_ref, out_ref):
    out_ref[0] = inp_ref[0] * emb_ref[...]


def kernel(inp, emb):
    grid = (S // S_BLK, B)
    return pl.pallas_call(
        _body,
        grid=grid,
        in_specs=[
            pl.BlockSpec((1, S_BLK, 1), lambda i, j: (j, i, 0)),
            pl.BlockSpec((S_BLK, D), lambda i, j: (i, 0)),
        ],
        out_specs=pl.BlockSpec((1, S_BLK, D), lambda i, j: (j, i, 0)),
        out_shape=jax.ShapeDtypeStruct((B, S, D), jnp.float32),
    )(inp, emb)
